# baseline (speedup 1.0000x reference)
import jax
import jax.numpy as jnp
from jax.experimental import pallas as pl

N_NODES = 100000
HEADS = 2
HID = 16
OUT_DIM = 32


def _gat_layer(h, src, dst, W, a_src, a_dst, b, heads, out_ch, concat, num_nodes):
    hp = (h @ W).reshape(num_nodes, heads, out_ch)
    alpha_src = (hp * a_src).sum(-1)
    alpha_dst = (hp * a_dst).sum(-1)
    alpha = alpha_src[src] + alpha_dst[dst]
    alpha = jax.nn.leaky_relu(alpha, 0.2)
    amax = jax.ops.segment_max(alpha, dst, num_segments=num_nodes)
    amax = jnp.where(jnp.isfinite(amax), amax, 0.0)
    ex = jnp.exp(alpha - amax[dst])
    denom = jax.ops.segment_sum(ex, dst, num_segments=num_nodes)
    coef = ex / (denom[dst] + 1e-16)
    msg = hp[src] * coef[:, :, None]
    out = jax.ops.segment_sum(msg, dst, num_segments=num_nodes)
    if concat:
        out = out.reshape(num_nodes, heads * out_ch)
    else:
        out = out.mean(axis=1)
    return out + b


def _addb(h_ref, b_ref, o_ref):
    o_ref[...] = h_ref[...] + b_ref[...]


def kernel(x, edge_index, emb, W1, att_src1, att_dst1, b1, W2, att_src2, att_dst2, b2):
    node_ids = x.squeeze()
    h = jnp.take(emb, node_ids, axis=0)
    loop = jnp.arange(N_NODES, dtype=edge_index.dtype)
    src = jnp.concatenate([edge_index[0], loop])
    dst = jnp.concatenate([edge_index[1], loop])
    h = _gat_layer(h, src, dst, W1, att_src1, att_dst1, b1, HEADS, HID, True, N_NODES)
    h = jax.nn.relu(h)
    h = _gat_layer(h, src, dst, W2, att_src2, att_dst2, jnp.zeros_like(b2), 1, OUT_DIM, False, N_NODES)
    out = pl.pallas_call(
        _addb,
        out_shape=jax.ShapeDtypeStruct((N_NODES, OUT_DIM), jnp.float32),
        grid=(N_NODES // 1000,),
        in_specs=[
            pl.BlockSpec((1000, OUT_DIM), lambda i: (i, 0)),
            pl.BlockSpec((1, OUT_DIM), lambda i: (0, 0)),
        ],
        out_specs=pl.BlockSpec((1000, OUT_DIM), lambda i: (i, 0)),
    )(h, b2.reshape(1, OUT_DIM))
    return out


# speedup vs baseline: 40.2238x; 40.2238x over previous
import functools

import jax
import jax.numpy as jnp
from jax import lax
from jax.experimental import pallas as pl
from jax.experimental.pallas import tpu as pltpu
from jax.experimental.pallas import tpu_sc as plsc

N_NODES = 100000
N_EDGES = 1600000
IN_DIM = 16
HID = 16
HEADS = 2
OUT_DIM = 32

NC, NS = 2, 16
NW = NC * NS
CHUNK = 128
N_PAD = 102400
E_TOT = N_EDGES + N_NODES
E_PAD = ((E_TOT + NW * CHUNK - 1) // (NW * CHUNK)) * (NW * CHUNK)
EDGES_PER_TILE = E_PAD // NS
CHUNKS_PER_TILE = EDGES_PER_TILE // CHUNK
ROWS_PER_TILE = N_PAD // NW
N_ACC = 100096
ROWS_ACC = N_ACC // NS
ZROWS = 391
Z1N = ROWS_ACC


def _mesh():
    return plsc.VectorSubcoreMesh(core_axis_name="c", subcore_axis_name="s")


def _emb_gather_body(ids_hbm, emb_hbm, h_hbm, idx_v, rows_v, sem):
    wid = lax.axis_index("s") * NC + lax.axis_index("c")
    base = wid * (N_PAD // NW)
    pltpu.sync_copy(ids_hbm.at[pl.ds(base, N_PAD // NW)], idx_v)
    n_chunks = (N_PAD // NW) // CHUNK

    def step(cix, _):
        off = cix * CHUNK
        pltpu.async_copy(
            emb_hbm.at[idx_v.at[pl.ds(off, CHUNK)]],
            rows_v.at[pl.ds(off, CHUNK)],
            sem,
        ).wait()
        return _

    lax.fori_loop(0, n_chunks, step, 0)
    pltpu.sync_copy(rows_v, h_hbm.at[pl.ds(base, N_PAD // NW)])


def _emb_gather(ids_pad, emb):
    return pl.kernel(
        _emb_gather_body,
        out_type=jax.ShapeDtypeStruct((N_PAD, IN_DIM), jnp.float32),
        mesh=_mesh(),
        compiler_params=pltpu.CompilerParams(use_tc_tiling_on_sc=False, needs_layout_passes=False),
        scratch_types=[
            pltpu.VMEM((N_PAD // NW,), jnp.int32),
            pltpu.VMEM((N_PAD // NW, IN_DIM), jnp.float32),
            pltpu.SemaphoreType.DMA,
        ],
    )(ids_pad, emb)


def _prep1_body(h_ref, w1_ref, as_ref, ad_ref, hp_ref, a_s_ref, a_d_ref):
    hp = jnp.dot(h_ref[...], w1_ref[...], preferred_element_type=jnp.float32)
    hp0 = hp[:, :HID]
    hp1 = hp[:, HID:]
    hp_ref[0] = hp0
    hp_ref[1] = hp1
    a_s_ref[0, :] = jnp.sum(hp0 * as_ref[0:1, :], axis=1)
    a_s_ref[1, :] = jnp.sum(hp1 * as_ref[1:2, :], axis=1)
    a_d_ref[0, :] = jnp.sum(hp0 * ad_ref[0:1, :], axis=1)
    a_d_ref[1, :] = jnp.sum(hp1 * ad_ref[1:2, :], axis=1)


def _prep1(h, W1, a_src, a_dst):
    blk = 1024
    return pl.pallas_call(
        _prep1_body,
        grid=(N_PAD // blk,),
        in_specs=[
            pl.BlockSpec((blk, IN_DIM), lambda i: (i, 0)),
            pl.BlockSpec((IN_DIM, HEADS * HID), lambda i: (0, 0)),
            pl.BlockSpec((HEADS, HID), lambda i: (0, 0)),
            pl.BlockSpec((HEADS, HID), lambda i: (0, 0)),
        ],
        out_specs=[
            pl.BlockSpec((HEADS, blk, HID), lambda i: (0, i, 0)),
            pl.BlockSpec((HEADS, blk), lambda i: (0, i)),
            pl.BlockSpec((HEADS, blk), lambda i: (0, i)),
        ],
        out_shape=[
            jax.ShapeDtypeStruct((HEADS, N_PAD, HID), jnp.float32),
            jax.ShapeDtypeStruct((HEADS, N_PAD), jnp.float32),
            jax.ShapeDtypeStruct((HEADS, N_PAD), jnp.float32),
        ],
    )(h, W1, a_src, a_dst)


def _edge_body(src_hbm, dst_hbm, tbl_hbm, as_hbm, ad_hbm, acc_hbm, den_hbm,
               sidx, didx, rows, wbuf, z2d, z1d,
               acc_s, den_s, ad_s, sem_r, sem_a, sem_d):
    cc = lax.axis_index("c")
    sid = lax.axis_index("s")
    wid = sid * NC + cc
    rbase = sid * ROWS_ACC

    def zstep(i, _):
        z2d[i, :] = jnp.zeros((HID,), jnp.float32)
        return _

    lax.fori_loop(0, ZROWS, zstep, 0)

    def z1step(i, _):
        z1d[pl.ds(i * 16, 16)] = jnp.zeros((16,), jnp.float32)
        return _

    lax.fori_loop(0, Z1N // 16, z1step, 0)

    def accz(i, _):
        pltpu.sync_copy(z2d, acc_s.at[pl.ds(rbase + i * ZROWS, ZROWS)])
        return _

    lax.fori_loop(0, ROWS_ACC // ZROWS, accz, 0)
    pltpu.sync_copy(z1d.at[pl.ds(0, ROWS_ACC)], den_s.at[pl.ds(rbase, ROWS_ACC)])

    pltpu.sync_copy(ad_hbm.at[cc].at[pl.ds(rbase, ROWS_ACC)],
                    ad_s.at[pl.ds(rbase, ROWS_ACC)])
    plsc.subcore_barrier()

    ebase = sid * EDGES_PER_TILE

    def chunk_step(cix, _):
        off = ebase + cix * CHUNK
        pltpu.sync_copy(src_hbm.at[pl.ds(off, CHUNK)], sidx)
        pltpu.sync_copy(dst_hbm.at[pl.ds(off, CHUNK)], didx)
        cp_r = pltpu.async_copy(tbl_hbm.at[cc].at[sidx], rows, sem_r)
        cp_a = pltpu.async_copy(as_hbm.at[cc].at[sidx], wbuf, sem_a)
        cp_d = pltpu.async_copy(ad_s.at[didx], z1d.at[pl.ds(0, CHUNK)], sem_d)
        cp_a.wait()
        cp_d.wait()

        def wstep(i, _):
            a = wbuf[pl.ds(i * 16, 16)] + z1d[pl.ds(i * 16, 16)]
            a = jnp.where(a > 0.0, a, 0.2 * a)
            eid = off + i * 16 + lax.iota(jnp.int32, 16)
            w = jnp.where(eid < E_TOT, jnp.exp(a), 0.0)
            wbuf[pl.ds(i * 16, 16)] = w
            return _

        lax.fori_loop(0, CHUNK // 16, wstep, 0)
        cp_r.wait()

        def estep(e, _):
            wv = plsc.load_gather(wbuf, [jnp.full((16,), e, jnp.int32)])
            rows[e, :] = rows[e, :] * wv
            return _

        lax.fori_loop(0, CHUNK, estep, 0)
        pltpu.sync_copy(rows, acc_s.at[didx], add=True)
        pltpu.sync_copy(wbuf, den_s.at[didx], add=True)
        return _

    lax.fori_loop(0, CHUNKS_PER_TILE, chunk_step, 0)
    plsc.subcore_barrier()

    pltpu.sync_copy(acc_s.at[pl.ds(rbase, ROWS_ACC)],
                    acc_hbm.at[cc].at[pl.ds(rbase, ROWS_ACC)])
    pltpu.sync_copy(den_s.at[pl.ds(rbase, ROWS_ACC)],
                    den_hbm.at[cc].at[pl.ds(rbase, ROWS_ACC)])


def _edge_pass(src_pad, dst_pad, tbl, a_s, a_d):
    return pl.kernel(
        _edge_body,
        out_type=[
            jax.ShapeDtypeStruct((NC, N_PAD, HID), jnp.float32),
            jax.ShapeDtypeStruct((NC, N_PAD), jnp.float32),
        ],
        mesh=_mesh(),
        compiler_params=pltpu.CompilerParams(use_tc_tiling_on_sc=False, needs_layout_passes=False),
        scratch_types=[
            pltpu.VMEM((CHUNK,), jnp.int32),
            pltpu.VMEM((CHUNK,), jnp.int32),
            pltpu.VMEM((CHUNK, HID), jnp.float32),
            pltpu.VMEM((CHUNK,), jnp.float32),
            pltpu.VMEM((ZROWS, HID), jnp.float32),
            pltpu.VMEM((Z1N,), jnp.float32),
            pltpu.VMEM_SHARED((N_ACC, HID), jnp.float32),
            pltpu.VMEM_SHARED((N_ACC,), jnp.float32),
            pltpu.VMEM_SHARED((N_ACC,), jnp.float32),
            pltpu.SemaphoreType.DMA,
            pltpu.SemaphoreType.DMA,
            pltpu.SemaphoreType.DMA,
        ],
    )(src_pad, dst_pad, tbl, a_s, a_d)


def _mid_body(acc_ref, den_ref, b1_ref, w2_ref, as_ref, ad_ref,
              hp_ref, a_s_ref, a_d_ref):
    d0 = den_ref[0, :][:, None] + 1e-16
    d1 = den_ref[1, :][:, None] + 1e-16
    h2 = jnp.concatenate([acc_ref[0] / d0, acc_ref[1] / d1], axis=1)
    h2 = jax.nn.relu(h2 + b1_ref[0:1, :])
    hp2 = jnp.dot(h2, w2_ref[...], preferred_element_type=jnp.float32)
    hp_ref[0] = hp2[:, :HID]
    hp_ref[1] = hp2[:, HID:]
    a_s = jnp.sum(hp2 * as_ref[...], axis=1)
    a_d = jnp.sum(hp2 * ad_ref[...], axis=1)
    a_s_ref[0, :] = a_s
    a_s_ref[1, :] = a_s
    a_d_ref[0, :] = a_d
    a_d_ref[1, :] = a_d


def _mid(acc1, den1, b1, W2, a_src2, a_dst2):
    blk = 1024
    return pl.pallas_call(
        _mid_body,
        grid=(N_PAD // blk,),
        in_specs=[
            pl.BlockSpec((NC, blk, HID), lambda i: (0, i, 0)),
            pl.BlockSpec((NC, blk), lambda i: (0, i)),
            pl.BlockSpec((1, HEADS * HID), lambda i: (0, 0)),
            pl.BlockSpec((HEADS * HID, OUT_DIM), lambda i: (0, 0)),
            pl.BlockSpec((1, OUT_DIM), lambda i: (0, 0)),
            pl.BlockSpec((1, OUT_DIM), lambda i: (0, 0)),
        ],
        out_specs=[
            pl.BlockSpec((NC, blk, HID), lambda i: (0, i, 0)),
            pl.BlockSpec((NC, blk), lambda i: (0, i)),
            pl.BlockSpec((NC, blk), lambda i: (0, i)),
        ],
        out_shape=[
            jax.ShapeDtypeStruct((NC, N_PAD, HID), jnp.float32),
            jax.ShapeDtypeStruct((NC, N_PAD), jnp.float32),
            jax.ShapeDtypeStruct((NC, N_PAD), jnp.float32),
        ],
    )(acc1, den1, b1, W2, a_src2, a_dst2)


def _final_body(acc_ref, den_ref, b2_ref, o_ref):
    d = den_ref[0, :, :] + 1e-16
    o = jnp.concatenate([acc_ref[0], acc_ref[1]], axis=1) / d
    o_ref[...] = o + b2_ref[0:1, :]


def _final(acc2, den2, b2):
    blk = 1000
    den2 = den2.reshape(NC, N_PAD, 1)
    return pl.pallas_call(
        _final_body,
        grid=(N_NODES // blk,),
        in_specs=[
            pl.BlockSpec((NC, blk, HID), lambda i: (0, i, 0)),
            pl.BlockSpec((NC, blk, 1), lambda i: (0, i, 0)),
            pl.BlockSpec((1, OUT_DIM), lambda i: (0, 0)),
        ],
        out_specs=pl.BlockSpec((blk, OUT_DIM), lambda i: (i, 0)),
        out_shape=jax.ShapeDtypeStruct((N_NODES, OUT_DIM), jnp.float32),
    )(acc2, den2, b2)


def kernel(x, edge_index, emb, W1, att_src1, att_dst1, b1, W2, att_src2,
           att_dst2, b2):
    ids = x.reshape(-1)
    ids_pad = jnp.concatenate(
        [ids, jnp.zeros((N_PAD - N_NODES,), jnp.int32)])
    loop = jnp.arange(N_NODES, dtype=jnp.int32)
    pad = jnp.zeros((E_PAD - E_TOT,), jnp.int32)
    src_pad = jnp.concatenate([edge_index[0], loop, pad])
    dst_pad = jnp.concatenate([edge_index[1], loop, pad])

    h = _emb_gather(ids_pad, emb)
    hp1, as1, ad1 = _prep1(h, W1, att_src1.reshape(HEADS, HID),
                           att_dst1.reshape(HEADS, HID))
    acc1, den1 = _edge_pass(src_pad, dst_pad, hp1, as1, ad1)
    hp2, as2, ad2 = _mid(acc1, den1, b1.reshape(1, HEADS * HID), W2,
                         att_src2.reshape(1, OUT_DIM),
                         att_dst2.reshape(1, OUT_DIM))
    acc2, den2 = _edge_pass(src_pad, dst_pad, hp2, as2, ad2)
    return _final(acc2, den2, b2.reshape(1, OUT_DIM))


# speedup vs baseline: 55.6905x; 1.3845x over previous
import functools

import jax
import jax.numpy as jnp
from jax import lax
from jax.experimental import pallas as pl
from jax.experimental.pallas import tpu as pltpu
from jax.experimental.pallas import tpu_sc as plsc

N_NODES = 100000
N_EDGES = 1600000
IN_DIM = 16
HID = 16
HEADS = 2
OUT_DIM = 32

NC, NS = 2, 16
NW = NC * NS
CHUNK = 128
N_PAD = 102400
E_TOT = N_EDGES + N_NODES
E_PAD = ((E_TOT + NW * CHUNK - 1) // (NW * CHUNK)) * (NW * CHUNK)
EDGES_PER_TILE = E_PAD // NS
CHUNKS_PER_TILE = EDGES_PER_TILE // CHUNK
ROWS_PER_TILE = N_PAD // NW
N_ACC = 100096
ROWS_ACC = N_ACC // NS
ZROWS = 391
Z1N = ROWS_ACC


def _mesh():
    return plsc.VectorSubcoreMesh(core_axis_name="c", subcore_axis_name="s")


def _emb_gather_body(ids_hbm, emb_hbm, h_hbm, idx_v, rows_v, sem):
    wid = lax.axis_index("s") * NC + lax.axis_index("c")
    base = wid * (N_PAD // NW)
    pltpu.sync_copy(ids_hbm.at[pl.ds(base, N_PAD // NW)], idx_v)
    n_chunks = (N_PAD // NW) // CHUNK

    def step(cix, _):
        off = cix * CHUNK
        pltpu.async_copy(
            emb_hbm.at[idx_v.at[pl.ds(off, CHUNK)]],
            rows_v.at[pl.ds(off, CHUNK)],
            sem,
        ).wait()
        return _

    lax.fori_loop(0, n_chunks, step, 0)
    pltpu.sync_copy(rows_v, h_hbm.at[pl.ds(base, N_PAD // NW)])


def _emb_gather(ids_pad, emb):
    return pl.kernel(
        _emb_gather_body,
        out_type=jax.ShapeDtypeStruct((N_PAD, IN_DIM), jnp.float32),
        mesh=_mesh(),
        compiler_params=pltpu.CompilerParams(use_tc_tiling_on_sc=False, needs_layout_passes=False),
        scratch_types=[
            pltpu.VMEM((N_PAD // NW,), jnp.int32),
            pltpu.VMEM((N_PAD // NW, IN_DIM), jnp.float32),
            pltpu.SemaphoreType.DMA,
        ],
    )(ids_pad, emb)


def _prep1_body(h_ref, w1_ref, as_ref, ad_ref, hp_ref, a_s_ref, a_d_ref):
    hp = jnp.dot(h_ref[...], w1_ref[...], preferred_element_type=jnp.float32)
    hp0 = hp[:, :HID]
    hp1 = hp[:, HID:]
    hp_ref[0] = hp0
    hp_ref[1] = hp1
    a_s_ref[0, :] = jnp.sum(hp0 * as_ref[0:1, :], axis=1)
    a_s_ref[1, :] = jnp.sum(hp1 * as_ref[1:2, :], axis=1)
    a_d_ref[0, :] = jnp.sum(hp0 * ad_ref[0:1, :], axis=1)
    a_d_ref[1, :] = jnp.sum(hp1 * ad_ref[1:2, :], axis=1)


def _prep1(h, W1, a_src, a_dst):
    blk = 1024
    return pl.pallas_call(
        _prep1_body,
        grid=(N_PAD // blk,),
        in_specs=[
            pl.BlockSpec((blk, IN_DIM), lambda i: (i, 0)),
            pl.BlockSpec((IN_DIM, HEADS * HID), lambda i: (0, 0)),
            pl.BlockSpec((HEADS, HID), lambda i: (0, 0)),
            pl.BlockSpec((HEADS, HID), lambda i: (0, 0)),
        ],
        out_specs=[
            pl.BlockSpec((HEADS, blk, HID), lambda i: (0, i, 0)),
            pl.BlockSpec((HEADS, blk), lambda i: (0, i)),
            pl.BlockSpec((HEADS, blk), lambda i: (0, i)),
        ],
        out_shape=[
            jax.ShapeDtypeStruct((HEADS, N_PAD, HID), jnp.float32),
            jax.ShapeDtypeStruct((HEADS, N_PAD), jnp.float32),
            jax.ShapeDtypeStruct((HEADS, N_PAD), jnp.float32),
        ],
    )(h, W1, a_src, a_dst)


def _edge_body(src_hbm, dst_hbm, tbl_hbm, as_hbm, ad_hbm, acc_hbm, den_hbm,
               sidx0, didx0, rows0, wbuf0, adv0,
               sidx1, didx1, rows1, wbuf1, adv1,
               z2d, z1d, acc_s, den_s, ad_s,
               sg_r0, sg_a0, sg_d0, sg_r1, sg_a1, sg_d1,
               ss_a0, ss_d0, ss_a1, ss_d1):
    cc = lax.axis_index("c")
    sid = lax.axis_index("s")
    rbase = sid * ROWS_ACC

    def zstep(i, _):
        z2d[i, :] = jnp.zeros((HID,), jnp.float32)
        return _

    lax.fori_loop(0, ZROWS, zstep, 0)

    def z1step(i, _):
        z1d[pl.ds(i * 16, 16)] = jnp.zeros((16,), jnp.float32)
        return _

    lax.fori_loop(0, Z1N // 16, z1step, 0)

    def accz(i, _):
        pltpu.sync_copy(z2d, acc_s.at[pl.ds(rbase + i * ZROWS, ZROWS)])
        return _

    lax.fori_loop(0, ROWS_ACC // ZROWS, accz, 0)
    pltpu.sync_copy(z1d.at[pl.ds(0, ROWS_ACC)], den_s.at[pl.ds(rbase, ROWS_ACC)])

    pltpu.sync_copy(ad_hbm.at[cc].at[pl.ds(rbase, ROWS_ACC)],
                    ad_s.at[pl.ds(rbase, ROWS_ACC)])
    plsc.subcore_barrier()

    ebase = sid * EDGES_PER_TILE
    bufs = [
        (sidx0, didx0, rows0, wbuf0, adv0, sg_r0, sg_a0, sg_d0, ss_a0, ss_d0),
        (sidx1, didx1, rows1, wbuf1, adv1, sg_r1, sg_a1, sg_d1, ss_a1, ss_d1),
    ]

    def issue(cix, b):
        sidx, didx, rows, wbuf, adv, sg_r, sg_a, sg_d, _, _ = bufs[b]
        off = ebase + cix * CHUNK
        pltpu.sync_copy(src_hbm.at[pl.ds(off, CHUNK)], sidx)
        pltpu.sync_copy(dst_hbm.at[pl.ds(off, CHUNK)], didx)
        pltpu.async_copy(tbl_hbm.at[cc].at[sidx], rows, sg_r)
        pltpu.async_copy(as_hbm.at[cc].at[sidx], wbuf, sg_a)
        pltpu.async_copy(ad_s.at[didx], adv, sg_d)

    def wait_gathers(b):
        sidx, didx, rows, wbuf, adv, sg_r, sg_a, sg_d, _, _ = bufs[b]
        pltpu.make_async_copy(tbl_hbm.at[cc].at[sidx], rows, sg_r).wait()
        pltpu.make_async_copy(as_hbm.at[cc].at[sidx], wbuf, sg_a).wait()
        pltpu.make_async_copy(as_hbm.at[cc].at[didx], adv, sg_d).wait()

    def wait_scatters(b):
        sidx, didx, rows, wbuf, adv, _, _, _, ss_a, ss_d = bufs[b]
        pltpu.make_async_copy(rows, acc_s.at[didx], ss_a).wait()
        pltpu.make_async_copy(wbuf, den_s.at[didx], ss_d).wait()

    def compute(cix, b):
        sidx, didx, rows, wbuf, adv, _, _, _, _, _ = bufs[b]
        off = ebase + cix * CHUNK

        def wstep(i, _):
            a = wbuf[pl.ds(i * 16, 16)] + adv[pl.ds(i * 16, 16)]
            a = jnp.where(a > 0.0, a, 0.2 * a)
            eid = off + i * 16 + lax.iota(jnp.int32, 16)
            w = jnp.where(eid < E_TOT, jnp.exp(a), 0.0)
            wbuf[pl.ds(i * 16, 16)] = w
            return _

        lax.fori_loop(0, CHUNK // 16, wstep, 0, unroll=4)

        def estep(e, _):
            wv = plsc.load_gather(wbuf, [jnp.full((16,), e, jnp.int32)])
            rows[e, :] = rows[e, :] * wv
            return _

        lax.fori_loop(0, CHUNK, estep, 0, unroll=8)

    def scatter(b):
        sidx, didx, rows, wbuf, adv, _, _, _, ss_a, ss_d = bufs[b]
        pltpu.async_copy(rows, acc_s.at[didx], ss_a, add=True)
        pltpu.async_copy(wbuf, den_s.at[didx], ss_d, add=True)

    issue(0, 0)

    def pair_step(j, carry):
        for ph in (0, 1):
            cix = 2 * j + ph
            nb = 1 - ph

            @pl.when(cix + 1 < CHUNKS_PER_TILE)
            def _():
                @pl.when(cix >= 1)
                def _():
                    wait_scatters(nb)
                issue(cix + 1, nb)

            wait_gathers(ph)
            compute(cix, ph)
            scatter(ph)
        return carry

    lax.fori_loop(0, CHUNKS_PER_TILE // 2, pair_step, 0)
    wait_scatters(0)
    wait_scatters(1)
    plsc.subcore_barrier()

    pltpu.sync_copy(acc_s.at[pl.ds(rbase, ROWS_ACC)],
                    acc_hbm.at[cc].at[pl.ds(rbase, ROWS_ACC)])
    pltpu.sync_copy(den_s.at[pl.ds(rbase, ROWS_ACC)],
                    den_hbm.at[cc].at[pl.ds(rbase, ROWS_ACC)])


def _edge_pass(src_pad, dst_pad, tbl, a_s, a_d):
    return pl.kernel(
        _edge_body,
        out_type=[
            jax.ShapeDtypeStruct((NC, N_PAD, HID), jnp.float32),
            jax.ShapeDtypeStruct((NC, N_PAD), jnp.float32),
        ],
        mesh=_mesh(),
        compiler_params=pltpu.CompilerParams(use_tc_tiling_on_sc=False, needs_layout_passes=False),
        scratch_types=[
            pltpu.VMEM((CHUNK,), jnp.int32),
            pltpu.VMEM((CHUNK,), jnp.int32),
            pltpu.VMEM((CHUNK, HID), jnp.float32),
            pltpu.VMEM((CHUNK,), jnp.float32),
            pltpu.VMEM((CHUNK,), jnp.float32),
            pltpu.VMEM((CHUNK,), jnp.int32),
            pltpu.VMEM((CHUNK,), jnp.int32),
            pltpu.VMEM((CHUNK, HID), jnp.float32),
            pltpu.VMEM((CHUNK,), jnp.float32),
            pltpu.VMEM((CHUNK,), jnp.float32),
            pltpu.VMEM((ZROWS, HID), jnp.float32),
            pltpu.VMEM((Z1N,), jnp.float32),
            pltpu.VMEM_SHARED((N_ACC, HID), jnp.float32),
            pltpu.VMEM_SHARED((N_ACC,), jnp.float32),
            pltpu.VMEM_SHARED((N_ACC,), jnp.float32),
        ] + [pltpu.SemaphoreType.DMA] * 10,
    )(src_pad, dst_pad, tbl, a_s, a_d)


def _mid_body(acc_ref, den_ref, b1_ref, w2_ref, as_ref, ad_ref,
              hp_ref, a_s_ref, a_d_ref):
    d0 = den_ref[0, :][:, None] + 1e-16
    d1 = den_ref[1, :][:, None] + 1e-16
    h2 = jnp.concatenate([acc_ref[0] / d0, acc_ref[1] / d1], axis=1)
    h2 = jax.nn.relu(h2 + b1_ref[0:1, :])
    hp2 = jnp.dot(h2, w2_ref[...], preferred_element_type=jnp.float32)
    hp_ref[0] = hp2[:, :HID]
    hp_ref[1] = hp2[:, HID:]
    a_s = jnp.sum(hp2 * as_ref[...], axis=1)
    a_d = jnp.sum(hp2 * ad_ref[...], axis=1)
    a_s_ref[0, :] = a_s
    a_s_ref[1, :] = a_s
    a_d_ref[0, :] = a_d
    a_d_ref[1, :] = a_d


def _mid(acc1, den1, b1, W2, a_src2, a_dst2):
    blk = 1024
    return pl.pallas_call(
        _mid_body,
        grid=(N_PAD // blk,),
        in_specs=[
            pl.BlockSpec((NC, blk, HID), lambda i: (0, i, 0)),
            pl.BlockSpec((NC, blk), lambda i: (0, i)),
            pl.BlockSpec((1, HEADS * HID), lambda i: (0, 0)),
            pl.BlockSpec((HEADS * HID, OUT_DIM), lambda i: (0, 0)),
            pl.BlockSpec((1, OUT_DIM), lambda i: (0, 0)),
            pl.BlockSpec((1, OUT_DIM), lambda i: (0, 0)),
        ],
        out_specs=[
            pl.BlockSpec((NC, blk, HID), lambda i: (0, i, 0)),
            pl.BlockSpec((NC, blk), lambda i: (0, i)),
            pl.BlockSpec((NC, blk), lambda i: (0, i)),
        ],
        out_shape=[
            jax.ShapeDtypeStruct((NC, N_PAD, HID), jnp.float32),
            jax.ShapeDtypeStruct((NC, N_PAD), jnp.float32),
            jax.ShapeDtypeStruct((NC, N_PAD), jnp.float32),
        ],
    )(acc1, den1, b1, W2, a_src2, a_dst2)


def _final_body(acc_ref, den_ref, b2_ref, o_ref):
    d = den_ref[0, :, :] + 1e-16
    o = jnp.concatenate([acc_ref[0], acc_ref[1]], axis=1) / d
    o_ref[...] = o + b2_ref[0:1, :]


def _final(acc2, den2, b2):
    blk = 1000
    den2 = den2.reshape(NC, N_PAD, 1)
    return pl.pallas_call(
        _final_body,
        grid=(N_NODES // blk,),
        in_specs=[
            pl.BlockSpec((NC, blk, HID), lambda i: (0, i, 0)),
            pl.BlockSpec((NC, blk, 1), lambda i: (0, i, 0)),
            pl.BlockSpec((1, OUT_DIM), lambda i: (0, 0)),
        ],
        out_specs=pl.BlockSpec((blk, OUT_DIM), lambda i: (i, 0)),
        out_shape=jax.ShapeDtypeStruct((N_NODES, OUT_DIM), jnp.float32),
    )(acc2, den2, b2)


def kernel(x, edge_index, emb, W1, att_src1, att_dst1, b1, W2, att_src2,
           att_dst2, b2):
    ids = x.reshape(-1)
    ids_pad = jnp.concatenate(
        [ids, jnp.zeros((N_PAD - N_NODES,), jnp.int32)])
    loop = jnp.arange(N_NODES, dtype=jnp.int32)
    pad = jnp.zeros((E_PAD - E_TOT,), jnp.int32)
    src_pad = jnp.concatenate([edge_index[0], loop, pad])
    dst_pad = jnp.concatenate([edge_index[1], loop, pad])

    h = _emb_gather(ids_pad, emb)
    hp1, as1, ad1 = _prep1(h, W1, att_src1.reshape(HEADS, HID),
                           att_dst1.reshape(HEADS, HID))
    acc1, den1 = _edge_pass(src_pad, dst_pad, hp1, as1, ad1)
    hp2, as2, ad2 = _mid(acc1, den1, b1.reshape(1, HEADS * HID), W2,
                         att_src2.reshape(1, OUT_DIM),
                         att_dst2.reshape(1, OUT_DIM))
    acc2, den2 = _edge_pass(src_pad, dst_pad, hp2, as2, ad2)
    return _final(acc2, den2, b2.reshape(1, OUT_DIM))


# speedup vs baseline: 57.7925x; 1.0377x over previous
import functools

import jax
import jax.numpy as jnp
from jax import lax
from jax.experimental import pallas as pl
from jax.experimental.pallas import tpu as pltpu
from jax.experimental.pallas import tpu_sc as plsc

N_NODES = 100000
N_EDGES = 1600000
IN_DIM = 16
HID = 16
HEADS = 2
OUT_DIM = 32

NC, NS = 2, 16
NW = NC * NS
CHUNK = 128
N_PAD = 102400
E_TOT = N_EDGES + N_NODES
E_PAD = ((E_TOT + NW * CHUNK - 1) // (NW * CHUNK)) * (NW * CHUNK)
EDGES_PER_TILE = E_PAD // NS
CHUNKS_PER_TILE = EDGES_PER_TILE // CHUNK
ROWS_PER_TILE = N_PAD // NW
N_ACC = 100096
ROWS_ACC = N_ACC // NS
ZROWS = 391
Z1N = ROWS_ACC


def _mesh():
    return plsc.VectorSubcoreMesh(core_axis_name="c", subcore_axis_name="s")


def _emb_gather_body(ids_hbm, emb_hbm, h_hbm, idx_v, rows_v, sem):
    wid = lax.axis_index("s") * NC + lax.axis_index("c")
    base = wid * (N_PAD // NW)
    pltpu.sync_copy(ids_hbm.at[pl.ds(base, N_PAD // NW)], idx_v)
    n_chunks = (N_PAD // NW) // CHUNK

    def step(cix, _):
        off = cix * CHUNK
        pltpu.async_copy(
            emb_hbm.at[idx_v.at[pl.ds(off, CHUNK)]],
            rows_v.at[pl.ds(off, CHUNK)],
            sem,
        ).wait()
        return _

    lax.fori_loop(0, n_chunks, step, 0)
    pltpu.sync_copy(rows_v, h_hbm.at[pl.ds(base, N_PAD // NW)])


def _emb_gather(ids_pad, emb):
    return pl.kernel(
        _emb_gather_body,
        out_type=jax.ShapeDtypeStruct((N_PAD, IN_DIM), jnp.float32),
        mesh=_mesh(),
        compiler_params=pltpu.CompilerParams(use_tc_tiling_on_sc=False, needs_layout_passes=False),
        scratch_types=[
            pltpu.VMEM((N_PAD // NW,), jnp.int32),
            pltpu.VMEM((N_PAD // NW, IN_DIM), jnp.float32),
            pltpu.SemaphoreType.DMA,
        ],
    )(ids_pad, emb)


def _prep1_body(h_ref, w1_ref, as_ref, ad_ref, hp_ref, a_s_ref, a_d_ref):
    hp = jnp.dot(h_ref[...], w1_ref[...], preferred_element_type=jnp.float32)
    hp0 = hp[:, :HID]
    hp1 = hp[:, HID:]
    hp_ref[0] = hp0
    hp_ref[1] = hp1
    a_s_ref[0, :] = jnp.sum(hp0 * as_ref[0:1, :], axis=1)
    a_s_ref[1, :] = jnp.sum(hp1 * as_ref[1:2, :], axis=1)
    a_d_ref[0, :] = jnp.sum(hp0 * ad_ref[0:1, :], axis=1)
    a_d_ref[1, :] = jnp.sum(hp1 * ad_ref[1:2, :], axis=1)


def _prep1(h, W1, a_src, a_dst):
    blk = 1024
    return pl.pallas_call(
        _prep1_body,
        grid=(N_PAD // blk,),
        in_specs=[
            pl.BlockSpec((blk, IN_DIM), lambda i: (i, 0)),
            pl.BlockSpec((IN_DIM, HEADS * HID), lambda i: (0, 0)),
            pl.BlockSpec((HEADS, HID), lambda i: (0, 0)),
            pl.BlockSpec((HEADS, HID), lambda i: (0, 0)),
        ],
        out_specs=[
            pl.BlockSpec((HEADS, blk, HID), lambda i: (0, i, 0)),
            pl.BlockSpec((HEADS, blk), lambda i: (0, i)),
            pl.BlockSpec((HEADS, blk), lambda i: (0, i)),
        ],
        out_shape=[
            jax.ShapeDtypeStruct((HEADS, N_PAD, HID), jnp.float32),
            jax.ShapeDtypeStruct((HEADS, N_PAD), jnp.float32),
            jax.ShapeDtypeStruct((HEADS, N_PAD), jnp.float32),
        ],
    )(h, W1, a_src, a_dst)


def _edge_body(sd_hbm, tbl_hbm, as_hbm, ad_hbm, acc_hbm, den_hbm,
               sd0, rows0, wbuf0, adv0,
               sd1, rows1, wbuf1, adv1,
               z2d, z1d, acc_s, den_s, ad_s,
               si0, sg_r0, sg_a0, sg_d0, si1, sg_r1, sg_a1, sg_d1,
               ss_a0, ss_d0, ss_a1, ss_d1):
    cc = lax.axis_index("c")
    sid = lax.axis_index("s")
    rbase = sid * ROWS_ACC

    def zstep(i, _):
        z2d[i, :] = jnp.zeros((HID,), jnp.float32)
        return _

    lax.fori_loop(0, ZROWS, zstep, 0)

    def z1step(i, _):
        z1d[pl.ds(i * 16, 16)] = jnp.zeros((16,), jnp.float32)
        return _

    lax.fori_loop(0, Z1N // 16, z1step, 0)

    def accz(i, _):
        pltpu.sync_copy(z2d, acc_s.at[pl.ds(rbase + i * ZROWS, ZROWS)])
        return _

    lax.fori_loop(0, ROWS_ACC // ZROWS, accz, 0)
    pltpu.sync_copy(z1d.at[pl.ds(0, ROWS_ACC)], den_s.at[pl.ds(rbase, ROWS_ACC)])

    pltpu.sync_copy(ad_hbm.at[cc].at[pl.ds(rbase, ROWS_ACC)],
                    ad_s.at[pl.ds(rbase, ROWS_ACC)])
    plsc.subcore_barrier()

    ebase = sid * EDGES_PER_TILE
    cbase = sid * CHUNKS_PER_TILE
    bufs = [
        (sd0, rows0, wbuf0, adv0, si0, sg_r0, sg_a0, sg_d0, ss_a0, ss_d0),
        (sd1, rows1, wbuf1, adv1, si1, sg_r1, sg_a1, sg_d1, ss_a1, ss_d1),
    ]

    def issue_idx(cix, b):
        sd, rows, wbuf, adv, si, sg_r, sg_a, sg_d, _, _ = bufs[b]
        pltpu.async_copy(sd_hbm.at[cbase + cix], sd, si)

    def wait_idx(b):
        sd, rows, wbuf, adv, si, sg_r, sg_a, sg_d, _, _ = bufs[b]
        pltpu.make_async_copy(sd_hbm.at[0], sd, si).wait()

    def issue_gathers(b):
        sd, rows, wbuf, adv, si, sg_r, sg_a, sg_d, _, _ = bufs[b]
        pltpu.async_copy(tbl_hbm.at[cc].at[sd.at[0]], rows, sg_r)
        pltpu.async_copy(as_hbm.at[cc].at[sd.at[0]], wbuf, sg_a)
        pltpu.async_copy(ad_s.at[sd.at[1]], adv, sg_d)

    def wait_gathers(b):
        sd, rows, wbuf, adv, si, sg_r, sg_a, sg_d, _, _ = bufs[b]
        pltpu.make_async_copy(tbl_hbm.at[cc].at[sd.at[0]], rows, sg_r).wait()
        pltpu.make_async_copy(as_hbm.at[cc].at[sd.at[0]], wbuf, sg_a).wait()
        pltpu.make_async_copy(as_hbm.at[cc].at[sd.at[1]], adv, sg_d).wait()

    def wait_scatters(b):
        sd, rows, wbuf, adv, _, _, _, _, ss_a, ss_d = bufs[b]
        pltpu.make_async_copy(rows, acc_s.at[sd.at[1]], ss_a).wait()
        pltpu.make_async_copy(wbuf, den_s.at[sd.at[1]], ss_d).wait()

    def compute(cix, b):
        sd, rows, wbuf, adv, _, _, _, _, _, _ = bufs[b]
        off = ebase + cix * CHUNK

        def wstep(i, _):
            a = wbuf[pl.ds(i * 16, 16)] + adv[pl.ds(i * 16, 16)]
            a = jnp.where(a > 0.0, a, 0.2 * a)
            eid = off + i * 16 + lax.iota(jnp.int32, 16)
            w = jnp.where(eid < E_TOT, jnp.exp(a), 0.0)
            wbuf[pl.ds(i * 16, 16)] = w
            return _

        lax.fori_loop(0, CHUNK // 16, wstep, 0, unroll=4)

        def estep(e, _):
            wv = plsc.load_gather(wbuf, [jnp.full((16,), e, jnp.int32)])
            rows[e, :] = rows[e, :] * wv
            return _

        lax.fori_loop(0, CHUNK, estep, 0, unroll=8)

    def scatter(b):
        sd, rows, wbuf, adv, _, _, _, _, ss_a, ss_d = bufs[b]
        pltpu.async_copy(rows, acc_s.at[sd.at[1]], ss_a, add=True)
        pltpu.async_copy(wbuf, den_s.at[sd.at[1]], ss_d, add=True)

    issue_idx(0, 0)
    wait_idx(0)
    issue_gathers(0)

    def pair_step(j, carry):
        for ph in (0, 1):
            cix = 2 * j + ph
            nb = 1 - ph

            @pl.when(cix + 1 < CHUNKS_PER_TILE)
            def _():
                @pl.when(cix >= 1)
                def _():
                    wait_scatters(nb)
                issue_idx(cix + 1, nb)

            wait_gathers(ph)
            compute(cix, ph)

            @pl.when(cix + 1 < CHUNKS_PER_TILE)
            def _():
                wait_idx(nb)
                issue_gathers(nb)

            scatter(ph)
        return carry

    lax.fori_loop(0, CHUNKS_PER_TILE // 2, pair_step, 0)
    wait_scatters(0)
    wait_scatters(1)
    plsc.subcore_barrier()

    pltpu.sync_copy(acc_s.at[pl.ds(rbase, ROWS_ACC)],
                    acc_hbm.at[cc].at[pl.ds(rbase, ROWS_ACC)])
    pltpu.sync_copy(den_s.at[pl.ds(rbase, ROWS_ACC)],
                    den_hbm.at[cc].at[pl.ds(rbase, ROWS_ACC)])


def _edge_pass(sd2d, tbl, a_s, a_d):
    return pl.kernel(
        _edge_body,
        out_type=[
            jax.ShapeDtypeStruct((NC, N_PAD, HID), jnp.float32),
            jax.ShapeDtypeStruct((NC, N_PAD), jnp.float32),
        ],
        mesh=_mesh(),
        compiler_params=pltpu.CompilerParams(use_tc_tiling_on_sc=False, needs_layout_passes=False),
        scratch_types=[
            pltpu.VMEM((2, CHUNK), jnp.int32),
            pltpu.VMEM((CHUNK, HID), jnp.float32),
            pltpu.VMEM((CHUNK,), jnp.float32),
            pltpu.VMEM((CHUNK,), jnp.float32),
            pltpu.VMEM((2, CHUNK), jnp.int32),
            pltpu.VMEM((CHUNK, HID), jnp.float32),
            pltpu.VMEM((CHUNK,), jnp.float32),
            pltpu.VMEM((CHUNK,), jnp.float32),
            pltpu.VMEM((ZROWS, HID), jnp.float32),
            pltpu.VMEM((Z1N,), jnp.float32),
            pltpu.VMEM_SHARED((N_ACC, HID), jnp.float32),
            pltpu.VMEM_SHARED((N_ACC,), jnp.float32),
            pltpu.VMEM_SHARED((N_ACC,), jnp.float32),
        ] + [pltpu.SemaphoreType.DMA] * 12,
    )(sd2d, tbl, a_s, a_d)


def _mid_body(acc_ref, den_ref, b1_ref, w2_ref, as_ref, ad_ref,
              hp_ref, a_s_ref, a_d_ref):
    d0 = den_ref[0, :][:, None] + 1e-16
    d1 = den_ref[1, :][:, None] + 1e-16
    h2 = jnp.concatenate([acc_ref[0] / d0, acc_ref[1] / d1], axis=1)
    h2 = jax.nn.relu(h2 + b1_ref[0:1, :])
    hp2 = jnp.dot(h2, w2_ref[...], preferred_element_type=jnp.float32)
    hp_ref[0] = hp2[:, :HID]
    hp_ref[1] = hp2[:, HID:]
    a_s = jnp.sum(hp2 * as_ref[...], axis=1)
    a_d = jnp.sum(hp2 * ad_ref[...], axis=1)
    a_s_ref[0, :] = a_s
    a_s_ref[1, :] = a_s
    a_d_ref[0, :] = a_d
    a_d_ref[1, :] = a_d


def _mid(acc1, den1, b1, W2, a_src2, a_dst2):
    blk = 1024
    return pl.pallas_call(
        _mid_body,
        grid=(N_PAD // blk,),
        in_specs=[
            pl.BlockSpec((NC, blk, HID), lambda i: (0, i, 0)),
            pl.BlockSpec((NC, blk), lambda i: (0, i)),
            pl.BlockSpec((1, HEADS * HID), lambda i: (0, 0)),
            pl.BlockSpec((HEADS * HID, OUT_DIM), lambda i: (0, 0)),
            pl.BlockSpec((1, OUT_DIM), lambda i: (0, 0)),
            pl.BlockSpec((1, OUT_DIM), lambda i: (0, 0)),
        ],
        out_specs=[
            pl.BlockSpec((NC, blk, HID), lambda i: (0, i, 0)),
            pl.BlockSpec((NC, blk), lambda i: (0, i)),
            pl.BlockSpec((NC, blk), lambda i: (0, i)),
        ],
        out_shape=[
            jax.ShapeDtypeStruct((NC, N_PAD, HID), jnp.float32),
            jax.ShapeDtypeStruct((NC, N_PAD), jnp.float32),
            jax.ShapeDtypeStruct((NC, N_PAD), jnp.float32),
        ],
    )(acc1, den1, b1, W2, a_src2, a_dst2)


def _final_body(acc_ref, den_ref, b2_ref, o_ref):
    d = den_ref[0, :, :] + 1e-16
    o = jnp.concatenate([acc_ref[0], acc_ref[1]], axis=1) / d
    o_ref[...] = o + b2_ref[0:1, :]


def _final(acc2, den2, b2):
    blk = 1000
    den2 = den2.reshape(NC, N_PAD, 1)
    return pl.pallas_call(
        _final_body,
        grid=(N_NODES // blk,),
        in_specs=[
            pl.BlockSpec((NC, blk, HID), lambda i: (0, i, 0)),
            pl.BlockSpec((NC, blk, 1), lambda i: (0, i, 0)),
            pl.BlockSpec((1, OUT_DIM), lambda i: (0, 0)),
        ],
        out_specs=pl.BlockSpec((blk, OUT_DIM), lambda i: (i, 0)),
        out_shape=jax.ShapeDtypeStruct((N_NODES, OUT_DIM), jnp.float32),
    )(acc2, den2, b2)


def kernel(x, edge_index, emb, W1, att_src1, att_dst1, b1, W2, att_src2,
           att_dst2, b2):
    ids = x.reshape(-1)
    ids_pad = jnp.concatenate(
        [ids, jnp.zeros((N_PAD - N_NODES,), jnp.int32)])
    loop = jnp.arange(N_NODES, dtype=jnp.int32)
    pad = jnp.zeros((E_PAD - E_TOT,), jnp.int32)
    src_pad = jnp.concatenate([edge_index[0], loop, pad])
    dst_pad = jnp.concatenate([edge_index[1], loop, pad])
    sd2d = jnp.stack([src_pad.reshape(E_PAD // CHUNK, CHUNK),
                      dst_pad.reshape(E_PAD // CHUNK, CHUNK)], axis=1)

    h = _emb_gather(ids_pad, emb)
    hp1, as1, ad1 = _prep1(h, W1, att_src1.reshape(HEADS, HID),
                           att_dst1.reshape(HEADS, HID))
    acc1, den1 = _edge_pass(sd2d, hp1, as1, ad1)
    hp2, as2, ad2 = _mid(acc1, den1, b1.reshape(1, HEADS * HID), W2,
                         att_src2.reshape(1, OUT_DIM),
                         att_dst2.reshape(1, OUT_DIM))
    acc2, den2 = _edge_pass(sd2d, hp2, as2, ad2)
    return _final(acc2, den2, b2.reshape(1, OUT_DIM))


# speedup vs baseline: 57.8173x; 1.0004x over previous
import functools

import jax
import jax.numpy as jnp
from jax import lax
from jax.experimental import pallas as pl
from jax.experimental.pallas import tpu as pltpu
from jax.experimental.pallas import tpu_sc as plsc

N_NODES = 100000
N_EDGES = 1600000
IN_DIM = 16
HID = 16
HEADS = 2
OUT_DIM = 32

NC, NS = 2, 16
NW = NC * NS
CHUNK = 128
N_PAD = 102400
E_TOT = N_EDGES + N_NODES
E_PAD = ((E_TOT + NW * CHUNK - 1) // (NW * CHUNK)) * (NW * CHUNK)
EDGES_PER_TILE = E_PAD // NS
CHUNKS_PER_TILE = EDGES_PER_TILE // CHUNK
ROWS_PER_TILE = N_PAD // NW
N_ACC = 100096
ROWS_ACC = N_ACC // NS
ZROWS = 391
Z1N = ROWS_ACC


def _mesh():
    return plsc.VectorSubcoreMesh(core_axis_name="c", subcore_axis_name="s")


def _emb_gather_body(ids_hbm, emb_hbm, h_hbm, idx_v, rows_v, sem):
    wid = lax.axis_index("s") * NC + lax.axis_index("c")
    base = wid * (N_PAD // NW)
    pltpu.sync_copy(ids_hbm.at[pl.ds(base, N_PAD // NW)], idx_v)
    n_chunks = (N_PAD // NW) // CHUNK

    def step(cix, _):
        off = cix * CHUNK
        pltpu.async_copy(
            emb_hbm.at[idx_v.at[pl.ds(off, CHUNK)]],
            rows_v.at[pl.ds(off, CHUNK)],
            sem,
        ).wait()
        return _

    lax.fori_loop(0, n_chunks, step, 0)
    pltpu.sync_copy(rows_v, h_hbm.at[pl.ds(base, N_PAD // NW)])


def _emb_gather(ids_pad, emb):
    return pl.kernel(
        _emb_gather_body,
        out_type=jax.ShapeDtypeStruct((N_PAD, IN_DIM), jnp.float32),
        mesh=_mesh(),
        compiler_params=pltpu.CompilerParams(use_tc_tiling_on_sc=False, needs_layout_passes=False),
        scratch_types=[
            pltpu.VMEM((N_PAD // NW,), jnp.int32),
            pltpu.VMEM((N_PAD // NW, IN_DIM), jnp.float32),
            pltpu.SemaphoreType.DMA,
        ],
    )(ids_pad, emb)


def _prep1_body(h_ref, w1_ref, as_ref, ad_ref, hp_ref, a_s_ref, a_d_ref):
    hp = jnp.dot(h_ref[...], w1_ref[...], preferred_element_type=jnp.float32)
    hp0 = hp[:, :HID]
    hp1 = hp[:, HID:]
    hp_ref[0] = hp0
    hp_ref[1] = hp1
    a_s_ref[0, :] = jnp.sum(hp0 * as_ref[0:1, :], axis=1)
    a_s_ref[1, :] = jnp.sum(hp1 * as_ref[1:2, :], axis=1)
    a_d_ref[0, :] = jnp.sum(hp0 * ad_ref[0:1, :], axis=1)
    a_d_ref[1, :] = jnp.sum(hp1 * ad_ref[1:2, :], axis=1)


def _prep1(h, W1, a_src, a_dst):
    blk = 1024
    return pl.pallas_call(
        _prep1_body,
        grid=(N_PAD // blk,),
        in_specs=[
            pl.BlockSpec((blk, IN_DIM), lambda i: (i, 0)),
            pl.BlockSpec((IN_DIM, HEADS * HID), lambda i: (0, 0)),
            pl.BlockSpec((HEADS, HID), lambda i: (0, 0)),
            pl.BlockSpec((HEADS, HID), lambda i: (0, 0)),
        ],
        out_specs=[
            pl.BlockSpec((HEADS, blk, HID), lambda i: (0, i, 0)),
            pl.BlockSpec((HEADS, blk), lambda i: (0, i)),
            pl.BlockSpec((HEADS, blk), lambda i: (0, i)),
        ],
        out_shape=[
            jax.ShapeDtypeStruct((HEADS, N_PAD, HID), jnp.float32),
            jax.ShapeDtypeStruct((HEADS, N_PAD), jnp.float32),
            jax.ShapeDtypeStruct((HEADS, N_PAD), jnp.float32),
        ],
    )(h, W1, a_src, a_dst)


def _edge_body(sd_hbm, tbl_hbm, as_hbm, ad_hbm, acc_hbm, den_hbm,
               sd0, rows0, wbuf0, adv0,
               sd1, rows1, wbuf1, adv1,
               z2d, z1d, acc_s, den_s, ad_s,
               si0, sg_r0, sg_a0, sg_d0, si1, sg_r1, sg_a1, sg_d1,
               ss_a0, ss_d0, ss_a1, ss_d1):
    cc = lax.axis_index("c")
    sid = lax.axis_index("s")
    rbase = sid * ROWS_ACC

    def zstep(i, _):
        z2d[i, :] = jnp.zeros((HID,), jnp.float32)
        return _

    lax.fori_loop(0, ZROWS, zstep, 0)

    def z1step(i, _):
        z1d[pl.ds(i * 16, 16)] = jnp.zeros((16,), jnp.float32)
        return _

    lax.fori_loop(0, Z1N // 16, z1step, 0)

    def accz(i, _):
        pltpu.sync_copy(z2d, acc_s.at[pl.ds(rbase + i * ZROWS, ZROWS)])
        return _

    lax.fori_loop(0, ROWS_ACC // ZROWS, accz, 0)
    pltpu.sync_copy(z1d.at[pl.ds(0, ROWS_ACC)], den_s.at[pl.ds(rbase, ROWS_ACC)])

    pltpu.sync_copy(ad_hbm.at[cc].at[pl.ds(rbase, ROWS_ACC)],
                    ad_s.at[pl.ds(rbase, ROWS_ACC)])
    plsc.subcore_barrier()

    ebase = sid * EDGES_PER_TILE
    cbase = sid * CHUNKS_PER_TILE
    bufs = [
        (sd0, rows0, wbuf0, adv0, si0, sg_r0, sg_a0, sg_d0, ss_a0, ss_d0),
        (sd1, rows1, wbuf1, adv1, si1, sg_r1, sg_a1, sg_d1, ss_a1, ss_d1),
    ]

    def issue_idx(cix, b):
        sd, rows, wbuf, adv, si, sg_r, sg_a, sg_d, _, _ = bufs[b]
        pltpu.async_copy(sd_hbm.at[pl.ds((cbase + cix) * 2, 2)], sd, si)

    def wait_idx(b):
        sd, rows, wbuf, adv, si, sg_r, sg_a, sg_d, _, _ = bufs[b]
        pltpu.make_async_copy(sd_hbm.at[pl.ds(0, 2)], sd, si).wait()

    def issue_gathers(b):
        sd, rows, wbuf, adv, si, sg_r, sg_a, sg_d, _, _ = bufs[b]
        pltpu.async_copy(tbl_hbm.at[cc].at[sd.at[0]], rows, sg_r)
        pltpu.async_copy(as_hbm.at[cc].at[sd.at[0]], wbuf, sg_a)
        pltpu.async_copy(ad_s.at[sd.at[1]], adv, sg_d)

    def wait_gathers(b):
        sd, rows, wbuf, adv, si, sg_r, sg_a, sg_d, _, _ = bufs[b]
        pltpu.make_async_copy(tbl_hbm.at[cc].at[sd.at[0]], rows, sg_r).wait()
        pltpu.make_async_copy(as_hbm.at[cc].at[sd.at[0]], wbuf, sg_a).wait()
        pltpu.make_async_copy(as_hbm.at[cc].at[sd.at[1]], adv, sg_d).wait()

    def wait_scatters(b):
        sd, rows, wbuf, adv, _, _, _, _, ss_a, ss_d = bufs[b]
        pltpu.make_async_copy(rows, acc_s.at[sd.at[1]], ss_a).wait()
        pltpu.make_async_copy(wbuf, den_s.at[sd.at[1]], ss_d).wait()

    def compute(cix, b):
        sd, rows, wbuf, adv, _, _, _, _, _, _ = bufs[b]
        off = ebase + cix * CHUNK

        def wstep(i, _):
            a = wbuf[pl.ds(i * 16, 16)] + adv[pl.ds(i * 16, 16)]
            a = jnp.where(a > 0.0, a, 0.2 * a)
            eid = off + i * 16 + lax.iota(jnp.int32, 16)
            w = jnp.where(eid < E_TOT, jnp.exp(a), 0.0)
            wbuf[pl.ds(i * 16, 16)] = w
            return _

        lax.fori_loop(0, CHUNK // 16, wstep, 0, unroll=4)

        def estep(e, _):
            wv = plsc.load_gather(wbuf, [jnp.full((16,), e, jnp.int32)])
            rows[e, :] = rows[e, :] * wv
            return _

        lax.fori_loop(0, CHUNK, estep, 0, unroll=8)

    def scatter(b):
        sd, rows, wbuf, adv, _, _, _, _, ss_a, ss_d = bufs[b]
        pltpu.async_copy(rows, acc_s.at[sd.at[1]], ss_a, add=True)
        pltpu.async_copy(wbuf, den_s.at[sd.at[1]], ss_d, add=True)

    issue_idx(0, 0)
    wait_idx(0)
    issue_gathers(0)

    def pair_step(j, carry):
        for ph in (0, 1):
            cix = 2 * j + ph
            nb = 1 - ph

            @pl.when(cix + 1 < CHUNKS_PER_TILE)
            def _():
                @pl.when(cix >= 1)
                def _():
                    wait_scatters(nb)
                issue_idx(cix + 1, nb)

            wait_gathers(ph)
            compute(cix, ph)

            @pl.when(cix + 1 < CHUNKS_PER_TILE)
            def _():
                wait_idx(nb)
                issue_gathers(nb)

            scatter(ph)
        return carry

    lax.fori_loop(0, CHUNKS_PER_TILE // 2, pair_step, 0)
    wait_scatters(0)
    wait_scatters(1)
    plsc.subcore_barrier()

    pltpu.sync_copy(acc_s.at[pl.ds(rbase, ROWS_ACC)],
                    acc_hbm.at[cc].at[pl.ds(rbase, ROWS_ACC)])
    pltpu.sync_copy(den_s.at[pl.ds(rbase, ROWS_ACC)],
                    den_hbm.at[cc].at[pl.ds(rbase, ROWS_ACC)])


def _edge_pass(sd2d, tbl, a_s, a_d):
    return pl.kernel(
        _edge_body,
        out_type=[
            jax.ShapeDtypeStruct((NC, N_PAD, HID), jnp.float32),
            jax.ShapeDtypeStruct((NC, N_PAD), jnp.float32),
        ],
        mesh=_mesh(),
        compiler_params=pltpu.CompilerParams(use_tc_tiling_on_sc=False, needs_layout_passes=False),
        scratch_types=[
            pltpu.VMEM((2, CHUNK), jnp.int32),
            pltpu.VMEM((CHUNK, HID), jnp.float32),
            pltpu.VMEM((CHUNK,), jnp.float32),
            pltpu.VMEM((CHUNK,), jnp.float32),
            pltpu.VMEM((2, CHUNK), jnp.int32),
            pltpu.VMEM((CHUNK, HID), jnp.float32),
            pltpu.VMEM((CHUNK,), jnp.float32),
            pltpu.VMEM((CHUNK,), jnp.float32),
            pltpu.VMEM((ZROWS, HID), jnp.float32),
            pltpu.VMEM((Z1N,), jnp.float32),
            pltpu.VMEM_SHARED((N_ACC, HID), jnp.float32),
            pltpu.VMEM_SHARED((N_ACC,), jnp.float32),
            pltpu.VMEM_SHARED((N_ACC,), jnp.float32),
        ] + [pltpu.SemaphoreType.DMA] * 12,
    )(sd2d, tbl, a_s, a_d)


def _mid_body(acc_ref, den_ref, b1_ref, w2_ref, as_ref, ad_ref,
              hp_ref, a_s_ref, a_d_ref):
    d0 = den_ref[0, :][:, None] + 1e-16
    d1 = den_ref[1, :][:, None] + 1e-16
    h2 = jnp.concatenate([acc_ref[0] / d0, acc_ref[1] / d1], axis=1)
    h2 = jax.nn.relu(h2 + b1_ref[0:1, :])
    hp2 = jnp.dot(h2, w2_ref[...], preferred_element_type=jnp.float32)
    hp_ref[0] = hp2[:, :HID]
    hp_ref[1] = hp2[:, HID:]
    a_s = jnp.sum(hp2 * as_ref[...], axis=1)
    a_d = jnp.sum(hp2 * ad_ref[...], axis=1)
    a_s_ref[0, :] = a_s
    a_s_ref[1, :] = a_s
    a_d_ref[0, :] = a_d
    a_d_ref[1, :] = a_d


def _mid(acc1, den1, b1, W2, a_src2, a_dst2):
    blk = 1024
    return pl.pallas_call(
        _mid_body,
        grid=(N_PAD // blk,),
        in_specs=[
            pl.BlockSpec((NC, blk, HID), lambda i: (0, i, 0)),
            pl.BlockSpec((NC, blk), lambda i: (0, i)),
            pl.BlockSpec((1, HEADS * HID), lambda i: (0, 0)),
            pl.BlockSpec((HEADS * HID, OUT_DIM), lambda i: (0, 0)),
            pl.BlockSpec((1, OUT_DIM), lambda i: (0, 0)),
            pl.BlockSpec((1, OUT_DIM), lambda i: (0, 0)),
        ],
        out_specs=[
            pl.BlockSpec((NC, blk, HID), lambda i: (0, i, 0)),
            pl.BlockSpec((NC, blk), lambda i: (0, i)),
            pl.BlockSpec((NC, blk), lambda i: (0, i)),
        ],
        out_shape=[
            jax.ShapeDtypeStruct((NC, N_PAD, HID), jnp.float32),
            jax.ShapeDtypeStruct((NC, N_PAD), jnp.float32),
            jax.ShapeDtypeStruct((NC, N_PAD), jnp.float32),
        ],
    )(acc1, den1, b1, W2, a_src2, a_dst2)


def _final_body(acc_ref, den_ref, b2_ref, o_ref):
    d = den_ref[0, :, :] + 1e-16
    o = jnp.concatenate([acc_ref[0], acc_ref[1]], axis=1) / d
    o_ref[...] = o + b2_ref[0:1, :]


def _final(acc2, den2, b2):
    blk = 1000
    den2 = den2.reshape(NC, N_PAD, 1)
    return pl.pallas_call(
        _final_body,
        grid=(N_NODES // blk,),
        in_specs=[
            pl.BlockSpec((NC, blk, HID), lambda i: (0, i, 0)),
            pl.BlockSpec((NC, blk, 1), lambda i: (0, i, 0)),
            pl.BlockSpec((1, OUT_DIM), lambda i: (0, 0)),
        ],
        out_specs=pl.BlockSpec((blk, OUT_DIM), lambda i: (i, 0)),
        out_shape=jax.ShapeDtypeStruct((N_NODES, OUT_DIM), jnp.float32),
    )(acc2, den2, b2)


def kernel(x, edge_index, emb, W1, att_src1, att_dst1, b1, W2, att_src2,
           att_dst2, b2):
    ids = x.reshape(-1)
    ids_pad = jnp.concatenate(
        [ids, jnp.zeros((N_PAD - N_NODES,), jnp.int32)])
    loop = jnp.arange(N_NODES, dtype=jnp.int32)
    pad = jnp.zeros((E_PAD - E_TOT,), jnp.int32)
    src_pad = jnp.concatenate([edge_index[0], loop, pad])
    dst_pad = jnp.concatenate([edge_index[1], loop, pad])
    sd2d = jnp.stack([src_pad.reshape(E_PAD // CHUNK, CHUNK),
                      dst_pad.reshape(E_PAD // CHUNK, CHUNK)],
                     axis=1).reshape(E_PAD // CHUNK * 2, CHUNK)

    h = _emb_gather(ids_pad, emb)
    hp1, as1, ad1 = _prep1(h, W1, att_src1.reshape(HEADS, HID),
                           att_dst1.reshape(HEADS, HID))
    acc1, den1 = _edge_pass(sd2d, hp1, as1, ad1)
    hp2, as2, ad2 = _mid(acc1, den1, b1.reshape(1, HEADS * HID), W2,
                         att_src2.reshape(1, OUT_DIM),
                         att_dst2.reshape(1, OUT_DIM))
    acc2, den2 = _edge_pass(sd2d, hp2, as2, ad2)
    return _final(acc2, den2, b2.reshape(1, OUT_DIM))
